# baseline (speedup 1.0000x reference)
import functools

import jax
import jax.numpy as jnp
from jax import lax
from jax.experimental import pallas as pl
from jax.experimental.pallas import tpu as pltpu
from jax.experimental.pallas import tpu_sc as plsc

_NR = 4
_NT = 4
_LA = 3


@functools.lru_cache(maxsize=None)
def _build_gather(V, D, B, L):
    info = plsc.get_sparse_core_info()
    NC, NS = info.num_cores, info.num_subcores
    NW = NC * NS
    LQ, DQ, BQ = L // 8, D // 8, B // 128
    assert L % _NR == 0 and L % 8 == 0 and D % 16 == 0 and BQ == NW
    n_groups = L // _NR
    mesh = plsc.VectorSubcoreMesh(core_axis_name="c", subcore_axis_name="s")

    @functools.partial(
        pl.kernel,
        mesh=mesh,
        compiler_params=pltpu.CompilerParams(use_tc_tiling_on_sc=False,
                                             needs_layout_passes=False),
        out_type=jax.ShapeDtypeStruct((L, DQ, BQ, 8, 128), jnp.float32),
        scratch_types=[
            pltpu.VMEM((LQ, 8, 128), jnp.int32),
            pltpu.VMEM((_NR, 128, D), jnp.float32),
            pltpu.VMEM((_NT, DQ, 8, 128), jnp.float32),
        ]
        + [pltpu.SemaphoreType.DMA] * (_NR + _NT),
    )
    def gather_kernel(idx_hbm, table_hbm, out_hbm, idx_v, rows, rows_t,
                      *sems):
        gs = sems[:_NR]
        os_ = sems[_NR:]
        wid = lax.axis_index("s") * NC + lax.axis_index("c")
        pltpu.sync_copy(idx_hbm.at[:, wid], idx_v)

        iota = lax.iota(jnp.int32, 16)
        diag = [(iota + i) % 16 for i in range(16)]

        def idx_row(l):
            return idx_v.at[l // 8, l % 8]

        def fire_gather(l, b):
            pltpu.async_copy(table_hbm.at[idx_row(l)], rows.at[b], gs[b])

        def wait_gather(l, b):
            pltpu.make_async_copy(table_hbm.at[idx_row(l)], rows.at[b],
                                  gs[b]).wait()

        def fire_out(l, bt):
            pltpu.async_copy(rows_t.at[bt], out_hbm.at[l, :, wid], os_[bt])

        def wait_out(l, bt):
            pltpu.make_async_copy(rows_t.at[bt], out_hbm.at[l, :, wid],
                                  os_[bt]).wait()

        def transpose(b, bt):
            src = rows.at[b]
            dst = rows_t.at[bt]

            def tb_body(tb2, carry):
                for h in range(2):
                    riv = iota + 32 * tb2 + 16 * h
                    for kb in range(D // 16):
                        vals = [plsc.load_gather(src,
                                                 [riv, diag[i] + 16 * kb])
                                for i in range(16)]
                        for i in range(16):
                            civ = diag[i] + 16 * kb
                            plsc.store_scatter(dst,
                                               [civ // 8, civ % 8, riv],
                                               vals[i])
                return carry

            lax.fori_loop(0, 128 // 32, tb_body, 0)

        for c in range(_LA):
            fire_gather(c, c % _NR)

        def group_body(g, carry):
            lg = g * _NR
            for j in range(_NR):
                l = lg + j
                b, bt = j % _NR, j % _NT

                @pl.when(l >= _NT)
                def _():
                    wait_out(l - _NT, bt)

                @pl.when(l + _LA < L)
                def _():
                    fire_gather(l + _LA, (j + _LA) % _NR)

                wait_gather(l, b)
                transpose(b, bt)
                fire_out(l, bt)
            return carry

        lax.fori_loop(0, n_groups, group_body, 0)

        for l in range(L - _NT, L):
            wait_out(l, l % _NT)

    def run(x5, table):
        return gather_kernel(x5, table)

    return run


def kernel(x, lens, embedding_weight):
    B, L = x.shape
    V, D = embedding_weight.shape
    run = _build_gather(V, D, B, L)
    x5 = x.astype(jnp.int32).reshape(B // 128, 128, L // 8, 8)
    x5 = x5.transpose(2, 0, 3, 1)
    out5 = run(x5, embedding_weight)
    return out5.transpose(2, 4, 0, 1, 3).reshape(B, L, D)


# speedup vs baseline: 1.1678x; 1.1678x over previous
import functools

import jax
import jax.numpy as jnp
from jax import lax
from jax.experimental import pallas as pl
from jax.experimental.pallas import tpu as pltpu
from jax.experimental.pallas import tpu_sc as plsc

_NR = 4
_NT = 4
_LA = 3


@functools.lru_cache(maxsize=None)
def _build_gather(V, D, B, L):
    info = plsc.get_sparse_core_info()
    NC, NS = info.num_cores, info.num_subcores
    NW = NC * NS
    LQ, DQ, BQ = L // 8, D // 8, B // 128
    assert L % _NR == 0 and L % 8 == 0 and D % 16 == 0 and BQ == NW
    n_groups = L // _NR
    mesh = plsc.VectorSubcoreMesh(core_axis_name="c", subcore_axis_name="s")

    @functools.partial(
        pl.kernel,
        mesh=mesh,
        compiler_params=pltpu.CompilerParams(use_tc_tiling_on_sc=False,
                                             needs_layout_passes=False),
        out_type=jax.ShapeDtypeStruct((L, DQ, BQ, 8, 128), jnp.float32),
        scratch_types=[
            pltpu.VMEM((LQ, 8, 128), jnp.int32),
            pltpu.VMEM((_NR, 128, D), jnp.float32),
            pltpu.VMEM((_NT, DQ, 8, 128), jnp.float32),
        ]
        + [pltpu.SemaphoreType.DMA] * (_NR + _NT),
    )
    def gather_kernel(idx_hbm, table_hbm, out_hbm, idx_v, rows, rows_t,
                      *sems):
        gs = sems[:_NR]
        os_ = sems[_NR:]
        wid = lax.axis_index("s") * NC + lax.axis_index("c")
        pltpu.sync_copy(idx_hbm.at[:, wid], idx_v)

        iota = lax.iota(jnp.int32, 16)
        diag = [(iota + i) % 16 for i in range(16)]

        def idx_row(l):
            return idx_v.at[l // 8, l % 8]

        def fire_gather(l, b):
            pltpu.async_copy(table_hbm.at[idx_row(l)], rows.at[b], gs[b])

        def wait_gather(l, b):
            pltpu.make_async_copy(table_hbm.at[idx_row(l)], rows.at[b],
                                  gs[b]).wait()

        def fire_out(l, bt):
            pltpu.async_copy(rows_t.at[bt], out_hbm.at[l, :, wid], os_[bt])

        def wait_out(l, bt):
            pltpu.make_async_copy(rows_t.at[bt], out_hbm.at[l, :, wid],
                                  os_[bt]).wait()

        def transpose(b, bt):
            src = rows.at[b]
            dst = rows_t.at[bt]

            def tb_body(tb, carry):
                riv = iota + 16 * tb
                for kb in range(D // 16):
                    vals = [plsc.load_gather(src, [riv, diag[i] + 16 * kb])
                            for i in range(16)]
                    for i in range(16):
                        civ = diag[i] + 16 * kb
                        plsc.store_scatter(dst, [civ // 8, civ % 8, riv],
                                           vals[i])
                return carry

            lax.fori_loop(0, 128 // 16, tb_body, 0)

        for c in range(_LA):
            fire_gather(c, c % _NR)

        def group_body(g, carry):
            lg = g * _NR
            for j in range(_NR):
                l = lg + j
                b, bt = j % _NR, j % _NT

                @pl.when(l >= _NT)
                def _():
                    wait_out(l - _NT, bt)

                @pl.when(l + _LA < L)
                def _():
                    fire_gather(l + _LA, (j + _LA) % _NR)

                wait_gather(l, b)
                transpose(b, bt)
                fire_out(l, bt)
            return carry

        lax.fori_loop(0, n_groups, group_body, 0)

        for l in range(L - _NT, L):
            wait_out(l, l % _NT)

    def run(x5, table):
        return gather_kernel(x5, table)

    return run


def kernel(x, lens, embedding_weight):
    B, L = x.shape
    V, D = embedding_weight.shape
    run = _build_gather(V, D, B, L)
    x5 = x.astype(jnp.int32).reshape(B // 128, 128, L // 8, 8)
    x5 = x5.transpose(2, 0, 3, 1)
    out5 = run(x5, embedding_weight)
    return out5.transpose(2, 4, 0, 1, 3).reshape(B, L, D)
